# baseline (speedup 1.0000x reference)
import functools

import numpy as np
import jax
import jax.numpy as jnp
from jax import lax
from jax.experimental import pallas as pl
from jax.experimental.pallas import tpu as pltpu
from jax.experimental.pallas import tpu_sc as plsc

D_MODEL = 768
N_VOCAB = 100000
SEQ = 4096
BATCH = 4
N_ROWS = SEQ * BATCH
SCALE = float(np.sqrt(np.float32(D_MODEL)))

NC, NS = 2, 16
NW = NC * NS
B_PER_W = N_ROWS // NW
CHUNK = 64
N_CHUNKS = B_PER_W // CHUNK
POS_PER_CHUNK = CHUNK // BATCH
LANES = 16
N_VEC = D_MODEL // LANES
NBUF = 2


@functools.lru_cache(maxsize=None)
def _positional_encoding():
    position = np.arange(0, SEQ, dtype=np.float32)[:, None]
    two_i = np.arange(0, D_MODEL, 2, dtype=np.float32)
    div_term = np.exp(two_i * -(np.log(10000.0) / D_MODEL))
    enc = np.zeros((SEQ, D_MODEL), dtype=np.float32)
    enc[:, 0::2] = np.sin(position * div_term)
    enc[:, 1::2] = np.cos(position * div_term)
    return enc


@functools.partial(
    pl.kernel,
    out_type=jax.ShapeDtypeStruct((N_ROWS, D_MODEL), jnp.float32),
    mesh=plsc.VectorSubcoreMesh(core_axis_name="c", subcore_axis_name="s"),
    scratch_types=[
        pltpu.VMEM((N_CHUNKS, CHUNK), jnp.int32),
        pltpu.VMEM((NBUF, CHUNK, D_MODEL), jnp.float32),
        pltpu.VMEM((NBUF, POS_PER_CHUNK, D_MODEL), jnp.float32),
        pltpu.SemaphoreType.DMA,
        pltpu.SemaphoreType.DMA,
    ],
)
def _emb_pe_kernel(x_hbm, w_hbm, pe_hbm, out_hbm, idx_v, emb_v, pe_v, gsem, osem):
    wid = lax.axis_index("s") * NC + lax.axis_index("c")
    pltpu.sync_copy(x_hbm.at[wid], idx_v)

    out_descs = [None] * N_CHUNKS
    gather_descs = [None] * N_CHUNKS
    gather_descs[0] = pltpu.async_copy(w_hbm.at[idx_v.at[0]], emb_v.at[0], gsem)

    for c in range(N_CHUNKS):
        p = c % NBUF
        if c + 1 < N_CHUNKS:
            if c >= 1:
                out_descs[c - 1].wait()
            gather_descs[c + 1] = pltpu.async_copy(
                w_hbm.at[idx_v.at[c + 1]], emb_v.at[(c + 1) % NBUF], gsem
            )
        pos0 = wid * (B_PER_W // BATCH) + c * POS_PER_CHUNK
        pltpu.sync_copy(pe_hbm.at[pl.ds(pos0, POS_PER_CHUNK)], pe_v.at[p])
        gather_descs[c].wait()

        def pos_body(pp, _):
            def col_body(j, _):
                off = j * LANES
                pe_vec = pe_v[p, pp, pl.ds(off, LANES)]
                for b in range(BATCH):
                    r = pp * BATCH + b
                    emb_v[p, r, pl.ds(off, LANES)] = (
                        emb_v[p, r, pl.ds(off, LANES)] * SCALE + pe_vec
                    )
                return 0

            return lax.fori_loop(0, N_VEC, col_body, 0)

        lax.fori_loop(0, POS_PER_CHUNK, pos_body, 0)

        row0 = wid * B_PER_W + c * CHUNK
        out_descs[c] = pltpu.async_copy(
            emb_v.at[p], out_hbm.at[pl.ds(row0, CHUNK)], osem
        )

    out_descs[N_CHUNKS - 2].wait()
    out_descs[N_CHUNKS - 1].wait()


def kernel(x, W):
    xf = x.astype(jnp.int32).reshape(NW, N_CHUNKS, CHUNK)
    pe = jnp.asarray(_positional_encoding())
    out = _emb_pe_kernel(xf, W, pe)
    return out.reshape(SEQ, BATCH, D_MODEL)


# speedup vs baseline: 1.2644x; 1.2644x over previous
import functools

import numpy as np
import jax
import jax.numpy as jnp
from jax import lax
from jax.experimental import pallas as pl
from jax.experimental.pallas import tpu as pltpu
from jax.experimental.pallas import tpu_sc as plsc

D_MODEL = 768
N_VOCAB = 100000
SEQ = 4096
BATCH = 4
N_ROWS = SEQ * BATCH
SCALE = float(np.sqrt(np.float32(D_MODEL)))

NC, NS = 2, 16
NW = NC * NS
B_PER_W = N_ROWS // NW
CHUNK = 64
N_CHUNKS = B_PER_W // CHUNK
POS_PER_CHUNK = CHUNK // BATCH
LANES = 16
N_VEC = D_MODEL // LANES
NBUF = 2


@functools.lru_cache(maxsize=None)
def _positional_encoding():
    position = np.arange(0, SEQ, dtype=np.float32)[:, None]
    two_i = np.arange(0, D_MODEL, 2, dtype=np.float32)
    div_term = np.exp(two_i * -(np.log(10000.0) / D_MODEL))
    enc = np.zeros((SEQ, D_MODEL), dtype=np.float32)
    enc[:, 0::2] = np.sin(position * div_term)
    enc[:, 1::2] = np.cos(position * div_term)
    return enc


@functools.partial(
    pl.kernel,
    out_type=jax.ShapeDtypeStruct((N_ROWS, D_MODEL), jnp.float32),
    mesh=plsc.VectorSubcoreMesh(core_axis_name="c", subcore_axis_name="s"),
    scratch_types=[
        pltpu.VMEM((N_CHUNKS, CHUNK), jnp.int32),
        pltpu.VMEM((NBUF, CHUNK, D_MODEL), jnp.float32),
        pltpu.VMEM((NBUF, POS_PER_CHUNK, D_MODEL), jnp.float32),
        pltpu.SemaphoreType.DMA,
        pltpu.SemaphoreType.DMA,
        pltpu.SemaphoreType.DMA,
    ],
)
def _emb_pe_kernel(
    x_hbm, w_hbm, pe_hbm, out_hbm, idx_v, emb_v, pe_v, gsem, psem, osem
):
    wid = lax.axis_index("s") * NC + lax.axis_index("c")
    pos_base = wid * (B_PER_W // BATCH)
    pltpu.sync_copy(x_hbm.at[wid], idx_v)

    def start_gather(c):
        return pltpu.async_copy(
            w_hbm.at[idx_v.at[c]], emb_v.at[c % NBUF], gsem
        )

    def start_pe(c):
        return pltpu.async_copy(
            pe_hbm.at[pl.ds(pos_base + c * POS_PER_CHUNK, POS_PER_CHUNK)],
            pe_v.at[c % NBUF],
            psem,
        )

    out_descs = [None] * N_CHUNKS
    gather_descs = [None] * N_CHUNKS
    pe_descs = [None] * N_CHUNKS
    gather_descs[0] = start_gather(0)
    pe_descs[0] = start_pe(0)

    for c in range(N_CHUNKS):
        p = c % NBUF
        if c + 1 < N_CHUNKS:
            if c >= 1:
                out_descs[c - 1].wait()
            gather_descs[c + 1] = start_gather(c + 1)
            pe_descs[c + 1] = start_pe(c + 1)
        pe_descs[c].wait()
        gather_descs[c].wait()

        def pos_body(pp, _):
            for j in range(N_VEC):
                off = j * LANES
                pe_vec = pe_v[p, pp, pl.ds(off, LANES)]
                for b in range(BATCH):
                    r = pp * BATCH + b
                    emb_v[p, r, pl.ds(off, LANES)] = (
                        emb_v[p, r, pl.ds(off, LANES)] * SCALE + pe_vec
                    )
            return 0

        lax.fori_loop(0, POS_PER_CHUNK, pos_body, 0)

        row0 = wid * B_PER_W + c * CHUNK
        out_descs[c] = pltpu.async_copy(
            emb_v.at[p], out_hbm.at[pl.ds(row0, CHUNK)], osem
        )

    out_descs[N_CHUNKS - 2].wait()
    out_descs[N_CHUNKS - 1].wait()


def kernel(x, W):
    xf = x.astype(jnp.int32).reshape(NW, N_CHUNKS, CHUNK)
    pe = jnp.asarray(_positional_encoding())
    out = _emb_pe_kernel(xf, W, pe)
    return out.reshape(SEQ, BATCH, D_MODEL)


# speedup vs baseline: 1.4746x; 1.1663x over previous
import functools

import numpy as np
import jax
import jax.numpy as jnp
from jax import lax
from jax.experimental import pallas as pl
from jax.experimental.pallas import tpu as pltpu
from jax.experimental.pallas import tpu_sc as plsc

D_MODEL = 768
N_VOCAB = 100000
SEQ = 4096
BATCH = 4
N_ROWS = SEQ * BATCH
SCALE = float(np.sqrt(np.float32(D_MODEL)))

NC, NS = 2, 16
NW = NC * NS
B_PER_W = N_ROWS // NW
CHUNK = 64
N_CHUNKS = B_PER_W // CHUNK
POS_PER_CHUNK = CHUNK // BATCH
LANES = 16
N_VEC = D_MODEL // LANES
NBUF = 2


@functools.lru_cache(maxsize=None)
def _positional_encoding():
    position = np.arange(0, SEQ, dtype=np.float32)[:, None]
    two_i = np.arange(0, D_MODEL, 2, dtype=np.float32)
    div_term = np.exp(two_i * -(np.log(10000.0) / D_MODEL))
    enc = np.zeros((SEQ, D_MODEL), dtype=np.float32)
    enc[:, 0::2] = np.sin(position * div_term)
    enc[:, 1::2] = np.cos(position * div_term)
    return enc


@functools.partial(
    pl.kernel,
    out_type=jax.ShapeDtypeStruct((N_ROWS, D_MODEL), jnp.float32),
    mesh=plsc.VectorSubcoreMesh(core_axis_name="c", subcore_axis_name="s"),
    scratch_types=[
        pltpu.VMEM((N_CHUNKS, CHUNK), jnp.int32),
        pltpu.VMEM((NBUF, CHUNK, D_MODEL), jnp.float32),
        pltpu.VMEM((NBUF, POS_PER_CHUNK, D_MODEL), jnp.float32),
        pltpu.SemaphoreType.DMA,
        pltpu.SemaphoreType.DMA,
        pltpu.SemaphoreType.DMA,
    ],
)
def _emb_pe_kernel(
    x_hbm, w_hbm, pe_hbm, out_hbm, idx_v, emb_v, pe_v, gsem, psem, osem
):
    wid = lax.axis_index("s") * NC + lax.axis_index("c")
    pos_base = wid * (B_PER_W // BATCH)
    row_base = wid * B_PER_W
    pltpu.sync_copy(x_hbm.at[wid], idx_v)

    def issue_fetch(c):
        b = lax.rem(c, NBUF)
        pltpu.async_copy(w_hbm.at[idx_v.at[c]], emb_v.at[b], gsem)
        pltpu.async_copy(
            pe_hbm.at[pl.ds(pos_base + c * POS_PER_CHUNK, POS_PER_CHUNK)],
            pe_v.at[b],
            psem,
        )

    def drain_out_one():
        pltpu.make_async_copy(
            emb_v.at[0], out_hbm.at[pl.ds(row_base, CHUNK)], osem
        ).wait()

    issue_fetch(0)

    @pl.loop(0, N_CHUNKS)
    def _chunk_loop(c):
        b = lax.rem(c, NBUF)

        @pl.when(c + 1 < N_CHUNKS)
        def _prefetch_next():
            @pl.when(c >= 1)
            def _free_buf():
                drain_out_one()

            issue_fetch(c + 1)

        pltpu.make_async_copy(w_hbm.at[idx_v.at[c]], emb_v.at[b], gsem).wait()
        pltpu.make_async_copy(
            pe_hbm.at[pl.ds(pos_base, POS_PER_CHUNK)], pe_v.at[b], psem
        ).wait()

        @plsc.parallel_loop(0, POS_PER_CHUNK)
        def _fma_pass(pp):
            for j in range(N_VEC):
                off = j * LANES
                pe_vec = pe_v[b, pp, pl.ds(off, LANES)]
                for bb in range(BATCH):
                    r = pp * BATCH + bb
                    emb_v[b, r, pl.ds(off, LANES)] = (
                        emb_v[b, r, pl.ds(off, LANES)] * SCALE + pe_vec
                    )

        pltpu.async_copy(
            emb_v.at[b], out_hbm.at[pl.ds(row_base + c * CHUNK, CHUNK)], osem
        )

    drain_out_one()
    drain_out_one()


def kernel(x, W):
    xf = x.astype(jnp.int32).reshape(NW, N_CHUNKS, CHUNK)
    pe = jnp.asarray(_positional_encoding())
    out = _emb_pe_kernel(xf, W, pe)
    return out.reshape(SEQ, BATCH, D_MODEL)


# speedup vs baseline: 1.4878x; 1.0090x over previous
import functools

import numpy as np
import jax
import jax.numpy as jnp
from jax import lax
from jax.experimental import pallas as pl
from jax.experimental.pallas import tpu as pltpu
from jax.experimental.pallas import tpu_sc as plsc

D_MODEL = 768
N_VOCAB = 100000
SEQ = 4096
BATCH = 4
N_ROWS = SEQ * BATCH
SCALE = float(np.sqrt(np.float32(D_MODEL)))

NC, NS = 2, 16
NW = NC * NS
B_PER_W = N_ROWS // NW
CHUNK = 32
N_CHUNKS = B_PER_W // CHUNK
POS_PER_CHUNK = CHUNK // BATCH
LANES = 16
N_VEC = D_MODEL // LANES
NBUF = 4
DRAIN_LAG = NBUF - 1


@functools.lru_cache(maxsize=None)
def _positional_encoding():
    position = np.arange(0, SEQ, dtype=np.float32)[:, None]
    two_i = np.arange(0, D_MODEL, 2, dtype=np.float32)
    div_term = np.exp(two_i * -(np.log(10000.0) / D_MODEL))
    enc = np.zeros((SEQ, D_MODEL), dtype=np.float32)
    enc[:, 0::2] = np.sin(position * div_term)
    enc[:, 1::2] = np.cos(position * div_term)
    return enc


@functools.partial(
    pl.kernel,
    out_type=jax.ShapeDtypeStruct((N_ROWS, D_MODEL), jnp.float32),
    mesh=plsc.VectorSubcoreMesh(core_axis_name="c", subcore_axis_name="s"),
    scratch_types=[
        pltpu.VMEM((N_CHUNKS, CHUNK), jnp.int32),
        pltpu.VMEM((NBUF, CHUNK, D_MODEL), jnp.float32),
        pltpu.VMEM((NBUF, POS_PER_CHUNK, D_MODEL), jnp.float32),
        pltpu.SemaphoreType.DMA,
        pltpu.SemaphoreType.DMA,
        pltpu.SemaphoreType.DMA,
    ],
)
def _emb_pe_kernel(
    x_hbm, w_hbm, pe_hbm, out_hbm, idx_v, emb_v, pe_v, gsem, psem, osem
):
    wid = lax.axis_index("s") * NC + lax.axis_index("c")
    pos_base = wid * (B_PER_W // BATCH)
    row_base = wid * B_PER_W
    pltpu.sync_copy(x_hbm.at[wid], idx_v)

    def issue_fetch(c):
        b = lax.rem(c, NBUF)
        pltpu.async_copy(w_hbm.at[idx_v.at[c]], emb_v.at[b], gsem)
        pltpu.async_copy(
            pe_hbm.at[pl.ds(pos_base + c * POS_PER_CHUNK, POS_PER_CHUNK)],
            pe_v.at[b],
            psem,
        )

    def drain_out_one():
        pltpu.make_async_copy(
            emb_v.at[0], out_hbm.at[pl.ds(row_base, CHUNK)], osem
        ).wait()

    issue_fetch(0)

    @pl.loop(0, N_CHUNKS)
    def _chunk_loop(c):
        b = lax.rem(c, NBUF)

        @pl.when(c + 1 < N_CHUNKS)
        def _prefetch_next():
            @pl.when(c >= DRAIN_LAG)
            def _free_buf():
                drain_out_one()

            issue_fetch(c + 1)

        pltpu.make_async_copy(w_hbm.at[idx_v.at[c]], emb_v.at[b], gsem).wait()
        pltpu.make_async_copy(
            pe_hbm.at[pl.ds(pos_base, POS_PER_CHUNK)], pe_v.at[b], psem
        ).wait()

        @plsc.parallel_loop(0, POS_PER_CHUNK)
        def _fma_pass(pp):
            for j in range(N_VEC):
                off = j * LANES
                pe_vec = pe_v[b, pp, pl.ds(off, LANES)]
                for bb in range(BATCH):
                    r = pp * BATCH + bb
                    emb_v[b, r, pl.ds(off, LANES)] = (
                        emb_v[b, r, pl.ds(off, LANES)] * SCALE + pe_vec
                    )

        pltpu.async_copy(
            emb_v.at[b], out_hbm.at[pl.ds(row_base + c * CHUNK, CHUNK)], osem
        )

    for _ in range(NBUF):
        drain_out_one()


def kernel(x, W):
    xf = x.astype(jnp.int32).reshape(NW, N_CHUNKS, CHUNK)
    pe = jnp.asarray(_positional_encoding())
    out = _emb_pe_kernel(xf, W, pe)
    return out.reshape(SEQ, BATCH, D_MODEL)
